# baseline (speedup 1.0000x reference)
import functools
import jax
import jax.numpy as jnp
from jax import lax
from jax.experimental import pallas as pl
from jax.experimental.pallas import tpu as pltpu
from jax.experimental.pallas import tpu_sc as plsc

_MAX_VALUE = 1000
_N_CHANNELS = 3
_EMBED_DIM = 32

_LANES = 16
_NUM_WORKERS = 32
_BCHUNK = 256
_PITCH = _BCHUNK + 8
_NBUF = 2


def _sc_body(n_units, n_batch, idx_hbm, table_hbm, out_hbm,
             table_v, idx_v, idx_s, buf0, buf1, sem0, sem1):
    wid = lax.axis_index("s") * 2 + lax.axis_index("c")

    per = n_units // _NUM_WORKERS
    rem = n_units - per * _NUM_WORKERS
    u_start = wid * per + jnp.minimum(wid, rem)
    u_end = u_start + per + jnp.where(wid < rem, 1, 0)

    pltpu.sync_copy(table_hbm, table_v)

    n_chunks = n_batch // _BCHUNK
    bufs = (buf0, buf1)
    sems = (sem0, sem1)
    row0 = lax.iota(jnp.int32, _LANES)
    row1 = row0 + _LANES

    def unit_body(u, carry):
        coff = (u % _N_CHANNELS) * _MAX_VALUE
        out_row0 = u * _EMBED_DIM
        pltpu.sync_copy(idx_hbm.at[pl.ds(u * n_batch, n_batch)], idx_v)

        def chunk_body(t2, carry2):
            for par in range(_NBUF):
                t = t2 * _NBUF + par
                b0 = t * _BCHUNK
                buf = bufs[par]
                @pl.when(t2 > 0)
                def _():
                    pltpu.make_async_copy(
                        buf.at[:, pl.ds(0, _BCHUNK)],
                        out_hbm.at[pl.ds(out_row0, _EMBED_DIM),
                                   pl.ds(b0, _BCHUNK)],
                        sems[par],
                    ).wait()

                def g_body(gi, carry3):
                    base = gi * _LANES
                    ivs = (idx_v[pl.ds(b0 + base, _LANES)] + coff) * _EMBED_DIM
                    basev = jnp.full((_LANES,), base, jnp.int32)
                    for k in range(0, _LANES, 4):
                        vals = []
                        for j in range(4):
                            pos = ivs[k + j]
                            vals.append(table_v[pl.ds(pos, _LANES)])
                            vals.append(table_v[pl.ds(pos + _LANES, _LANES)])
                        for j in range(4):
                            cv = basev + (k + j)
                            plsc.store_scatter(buf, [row0, cv], vals[2 * j])
                            plsc.store_scatter(buf, [row1, cv], vals[2 * j + 1])
                    return carry3

                lax.fori_loop(0, _BCHUNK // _LANES, g_body, 0)

                pltpu.async_copy(
                    buf.at[:, pl.ds(0, _BCHUNK)],
                    out_hbm.at[pl.ds(out_row0, _EMBED_DIM),
                               pl.ds(b0, _BCHUNK)],
                    sems[par],
                )
            return carry2

        lax.fori_loop(0, n_chunks // _NBUF, chunk_body, 0)
        for par in range(_NBUF):
            pltpu.make_async_copy(
                bufs[par].at[:, pl.ds(0, _BCHUNK)],
                out_hbm.at[pl.ds(out_row0, _EMBED_DIM), pl.ds(0, _BCHUNK)],
                sems[par],
            ).wait()
        return carry

    lax.fori_loop(u_start, u_end, unit_body, 0)


def kernel(inputs, table):
    b, h, w, ch = inputs.shape
    assert ch == _N_CHANNELS and table.shape == (_N_CHANNELS * _MAX_VALUE, _EMBED_DIM)
    n_units = h * w * ch
    idx_lin = jnp.transpose(inputs, (1, 2, 3, 0)).reshape(-1)
    table_flat = table.reshape(-1)

    mesh = plsc.VectorSubcoreMesh(core_axis_name="c", subcore_axis_name="s")
    sc_call = pl.kernel(
        functools.partial(_sc_body, n_units, b),
        out_type=jax.ShapeDtypeStruct((n_units * _EMBED_DIM, b), jnp.float32),
        mesh=mesh,
        scratch_types=[
            pltpu.VMEM((table.size,), jnp.float32),
            pltpu.VMEM((b,), jnp.int32),
            pltpu.SMEM((_BCHUNK,), jnp.int32),
            pltpu.VMEM((_EMBED_DIM, _PITCH), jnp.float32),
            pltpu.VMEM((_EMBED_DIM, _PITCH), jnp.float32),
            pltpu.SemaphoreType.DMA,
            pltpu.SemaphoreType.DMA,
        ],
        compiler_params=pltpu.CompilerParams(
            use_tc_tiling_on_sc=False, needs_layout_passes=False
        ),
    )
    out = sc_call(idx_lin, table_flat)
    return out.reshape(h, w, ch * _EMBED_DIM, b).transpose(3, 0, 1, 2)


# speedup vs baseline: 1.0046x; 1.0046x over previous
import functools
import jax
import jax.numpy as jnp
from jax import lax
from jax.experimental import pallas as pl
from jax.experimental.pallas import tpu as pltpu
from jax.experimental.pallas import tpu_sc as plsc

_MAX_VALUE = 1000
_N_CHANNELS = 3
_EMBED_DIM = 32

_LANES = 16
_NUM_WORKERS = 32
_BCHUNK = 256
_PITCH = _BCHUNK + 8
_NBUF = 2


def _sc_body(n_units, n_batch, idx_hbm, table_hbm, out_hbm,
             table_v, idx_v, idx_s, buf0, buf1, sem0, sem1):
    wid = lax.axis_index("s") * 2 + lax.axis_index("c")

    per = n_units // _NUM_WORKERS
    rem = n_units - per * _NUM_WORKERS
    u_start = wid * per + jnp.minimum(wid, rem)
    u_end = u_start + per + jnp.where(wid < rem, 1, 0)

    pltpu.sync_copy(table_hbm, table_v)

    n_chunks = n_batch // _BCHUNK
    bufs = (buf0, buf1)
    sems = (sem0, sem1)
    row0 = lax.iota(jnp.int32, _LANES)
    row1 = row0 + _LANES

    def unit_body(u, carry):
        coff = (u % _N_CHANNELS) * _MAX_VALUE
        out_row0 = u * _EMBED_DIM
        pltpu.sync_copy(idx_hbm.at[pl.ds(u * n_batch, n_batch)], idx_v)

        def chunk_body(t2, carry2):
            for par in range(_NBUF):
                t = t2 * _NBUF + par
                b0 = t * _BCHUNK
                buf = bufs[par]
                @pl.when(t2 > 0)
                def _():
                    pltpu.make_async_copy(
                        buf.at[:, pl.ds(0, _BCHUNK)],
                        out_hbm.at[pl.ds(out_row0, _EMBED_DIM),
                                   pl.ds(b0, _BCHUNK)],
                        sems[par],
                    ).wait()

                def g_body(gi, carry3):
                    base = gi * _LANES
                    ivs = (idx_v[pl.ds(b0 + base, _LANES)] + coff) * _EMBED_DIM
                    basev = jnp.full((_LANES,), base, jnp.int32)
                    for k in range(0, _LANES, 4):
                        vals = []
                        for j in range(4):
                            pos = ivs[k + j]
                            vals.append(table_v[pl.ds(pos, _LANES)])
                            vals.append(table_v[pl.ds(pos + _LANES, _LANES)])
                        for j in range(4):
                            cv = basev + (k + j)
                            plsc.store_scatter(buf, [row0, cv], vals[2 * j])
                            plsc.store_scatter(buf, [row1, cv], vals[2 * j + 1])
                    return carry3

                lax.fori_loop(0, _BCHUNK // _LANES, g_body, 0)

                pltpu.async_copy(
                    buf.at[:, pl.ds(0, _BCHUNK)],
                    out_hbm.at[pl.ds(out_row0, _EMBED_DIM),
                               pl.ds(b0, _BCHUNK)],
                    sems[par],
                )
            return carry2

        lax.fori_loop(0, n_chunks // _NBUF, chunk_body, 0)
        for par in range(_NBUF):
            pltpu.make_async_copy(
                bufs[par].at[:, pl.ds(0, _BCHUNK)],
                out_hbm.at[pl.ds(out_row0, _EMBED_DIM), pl.ds(0, _BCHUNK)],
                sems[par],
            ).wait()
        return carry

    lax.fori_loop(u_start, u_end, unit_body, 0)


def kernel(inputs, table):
    b, h, w, ch = inputs.shape
    assert ch == _N_CHANNELS and table.shape == (_N_CHANNELS * _MAX_VALUE, _EMBED_DIM)
    n_units = h * w * ch
    idx_lin = jnp.transpose(inputs, (1, 2, 3, 0)).reshape(-1)
    table_flat = table.reshape(-1)

    mesh = plsc.VectorSubcoreMesh(core_axis_name="c", subcore_axis_name="s")
    sc_call = pl.kernel(
        functools.partial(_sc_body, n_units, b),
        out_type=jax.ShapeDtypeStruct((n_units * _EMBED_DIM, b), jnp.float32),
        mesh=mesh,
        scratch_types=[
            pltpu.VMEM((table.size,), jnp.float32),
            pltpu.VMEM((b,), jnp.int32),
            pltpu.SMEM((_BCHUNK,), jnp.int32),
            pltpu.VMEM((_EMBED_DIM, _PITCH), jnp.float32),
            pltpu.VMEM((_EMBED_DIM, _PITCH), jnp.float32),
            pltpu.SemaphoreType.DMA,
            pltpu.SemaphoreType.DMA,
        ],
        compiler_params=pltpu.CompilerParams(
            use_tc_tiling_on_sc=False,
            needs_layout_passes=False,
            skip_device_barrier=True,
        ),
    )
    out = sc_call(idx_lin, table_flat)
    return out.reshape(h, w, ch * _EMBED_DIM, b).transpose(3, 0, 1, 2)


# speedup vs baseline: 1.8350x; 1.8266x over previous
import functools
import jax
import jax.numpy as jnp
from jax import lax
from jax.experimental import pallas as pl
from jax.experimental.pallas import tpu as pltpu
from jax.experimental.pallas import tpu_sc as plsc

_MAX_VALUE = 1000
_N_CHANNELS = 3
_EMBED_DIM = 32

_LANES = 16
_NUM_WORKERS = 32
_BCHUNK = 256
_PITCH = _BCHUNK + 8
_NBUF = 2
_BANDS = _EMBED_DIM // 8
_BTILES = _BCHUNK // 128
_TILES = _BANDS * _BTILES


def _sc_body(n_units, n_batch, idx_hbm, table_hbm, out_hbm,
             table_v, idx_v, buf0, buf1, sem0, sem1):
    wid = lax.axis_index("s") * 2 + lax.axis_index("c")

    per = n_units // _NUM_WORKERS
    rem = n_units - per * _NUM_WORKERS
    u_start = wid * per + jnp.minimum(wid, rem)
    u_end = u_start + per + jnp.where(wid < rem, 1, 0)

    pltpu.sync_copy(table_hbm, table_v)

    n_chunks = n_batch // _BCHUNK
    n_btiles = n_batch // 128
    bufs = (buf0, buf1)
    sems = (sem0, sem1)
    row0 = lax.iota(jnp.int32, _LANES)
    row1 = row0 + _LANES

    def _emit_tiles(buf, tile_row0, bt0, sem):
        for band in range(_BANDS):
            for btl in range(_BTILES):
                pltpu.async_copy(
                    buf.at[pl.ds(band * 8, 8), pl.ds(btl * 128, 128)],
                    out_hbm.at[tile_row0 + band * n_btiles + bt0 + btl],
                    sem,
                )

    def _drain_tiles(buf, tile_row0, sem):
        for _ in range(_TILES):
            pltpu.make_async_copy(
                buf.at[pl.ds(0, 8), pl.ds(0, 128)],
                out_hbm.at[tile_row0],
                sem,
            ).wait()

    def unit_body(u, carry):
        coff = (u % _N_CHANNELS) * _MAX_VALUE
        tile_row0 = u * _BANDS * n_btiles
        pltpu.sync_copy(idx_hbm.at[pl.ds(u * n_batch, n_batch)], idx_v)

        def chunk_body(t2, carry2):
            for par in range(_NBUF):
                t = t2 * _NBUF + par
                b0 = t * _BCHUNK
                buf = bufs[par]
                @pl.when(t2 > 0)
                def _():
                    _drain_tiles(buf, tile_row0, sems[par])

                def g_body(gi, carry3):
                    base = gi * _LANES
                    ivs = (idx_v[pl.ds(b0 + base, _LANES)] + coff) * _EMBED_DIM
                    basev = jnp.full((_LANES,), base, jnp.int32)
                    for k in range(0, _LANES, 4):
                        vals = []
                        for j in range(4):
                            pos = ivs[k + j]
                            vals.append(table_v[pl.ds(pos, _LANES)])
                            vals.append(table_v[pl.ds(pos + _LANES, _LANES)])
                        for j in range(4):
                            cv = basev + (k + j)
                            plsc.store_scatter(buf, [row0, cv], vals[2 * j])
                            plsc.store_scatter(buf, [row1, cv], vals[2 * j + 1])
                    return carry3

                lax.fori_loop(0, _BCHUNK // _LANES, g_body, 0)

                _emit_tiles(buf, tile_row0, t * _BTILES, sems[par])
            return carry2

        lax.fori_loop(0, n_chunks // _NBUF, chunk_body, 0)
        for par in range(_NBUF):
            _drain_tiles(bufs[par], tile_row0, sems[par])
        return carry

    lax.fori_loop(u_start, u_end, unit_body, 0)


def kernel(inputs, table):
    b, h, w, ch = inputs.shape
    assert ch == _N_CHANNELS and table.shape == (_N_CHANNELS * _MAX_VALUE, _EMBED_DIM)
    n_units = h * w * ch
    idx_lin = jnp.transpose(inputs, (1, 2, 3, 0)).reshape(-1)
    table_flat = table.reshape(-1)
    n_btiles = b // 128
    n_tile_rows = n_units * _BANDS

    mesh = plsc.VectorSubcoreMesh(core_axis_name="c", subcore_axis_name="s")
    sc_call = pl.kernel(
        functools.partial(_sc_body, n_units, b),
        out_type=jax.ShapeDtypeStruct((n_tile_rows * n_btiles, 8, 128),
                                      jnp.float32),
        mesh=mesh,
        scratch_types=[
            pltpu.VMEM((table.size,), jnp.float32),
            pltpu.VMEM((b,), jnp.int32),
            pltpu.VMEM((_EMBED_DIM, _PITCH), jnp.float32),
            pltpu.VMEM((_EMBED_DIM, _PITCH), jnp.float32),
            pltpu.SemaphoreType.DMA,
            pltpu.SemaphoreType.DMA,
        ],
        compiler_params=pltpu.CompilerParams(
            use_tc_tiling_on_sc=False,
            needs_layout_passes=False,
        ),
    )
    out = sc_call(idx_lin, table_flat)
    t = out.reshape(h, w, ch * _BANDS, n_btiles, 8, 128)
    t = t.transpose(3, 5, 0, 1, 2, 4)
    return t.reshape(b, h, w, ch * _EMBED_DIM)


# speedup vs baseline: 1.8837x; 1.0265x over previous
import functools
import jax
import jax.numpy as jnp
from jax import lax
from jax.experimental import pallas as pl
from jax.experimental.pallas import tpu as pltpu
from jax.experimental.pallas import tpu_sc as plsc

_MAX_VALUE = 1000
_N_CHANNELS = 3
_EMBED_DIM = 32

_LANES = 16
_NUM_WORKERS = 32
_BCHUNK = 256
_PITCH = _BCHUNK + 8
_NBUF = 2
_BANDS = _EMBED_DIM // 8
_BTILES = _BCHUNK // 128
_TILES = _BANDS * _BTILES


def _sc_body(n_units, n_batch, idx_hbm, table_hbm, out_hbm,
             table_v, idx_v, buf0, buf1, sem0, sem1):
    wid = lax.axis_index("s") * 2 + lax.axis_index("c")

    per = n_units // _NUM_WORKERS
    rem = n_units - per * _NUM_WORKERS
    u_start = wid * per + jnp.minimum(wid, rem)
    u_end = u_start + per + jnp.where(wid < rem, 1, 0)

    pltpu.sync_copy(table_hbm, table_v)

    n_chunks = n_batch // _BCHUNK
    n_btiles = n_batch // 128
    bufs = (buf0, buf1)
    sems = (sem0, sem1)
    row0 = lax.iota(jnp.int32, _LANES)
    row1 = row0 + _LANES

    def _emit_tiles(buf, tile_row0, bt0, sem):
        for band in range(_BANDS):
            for btl in range(_BTILES):
                pltpu.async_copy(
                    buf.at[pl.ds(band * 8, 8), pl.ds(btl * 128, 128)],
                    out_hbm.at[tile_row0 + band * n_btiles + bt0 + btl],
                    sem,
                )

    def _drain_tiles(buf, tile_row0, sem):
        for _ in range(_TILES):
            pltpu.make_async_copy(
                buf.at[pl.ds(0, 8), pl.ds(0, 128)],
                out_hbm.at[tile_row0],
                sem,
            ).wait()

    def unit_body(u, carry):
        coff = (u % _N_CHANNELS) * _MAX_VALUE
        tile_row0 = u * _BANDS * n_btiles
        pltpu.sync_copy(idx_hbm.at[pl.ds(u * n_batch, n_batch)], idx_v)

        def chunk_body(t2, carry2):
            for par in range(_NBUF):
                t = t2 * _NBUF + par
                b0 = t * _BCHUNK
                buf = bufs[par]
                @pl.when(t2 > 0)
                def _():
                    _drain_tiles(buf, tile_row0, sems[par])

                def g_body(gi, carry3):
                    base = gi * _LANES
                    ivs = (idx_v[pl.ds(b0 + base, _LANES)] + coff) * _EMBED_DIM
                    basev = jnp.full((_LANES,), base, jnp.int32)
                    for k in range(0, _LANES, 8):
                        vals = []
                        for j in range(8):
                            pos = ivs[k + j]
                            vals.append(table_v[pl.ds(pos, _LANES)])
                            vals.append(table_v[pl.ds(pos + _LANES, _LANES)])
                        for j in range(8):
                            cv = basev + (k + j)
                            plsc.store_scatter(buf, [row0, cv], vals[2 * j])
                            plsc.store_scatter(buf, [row1, cv], vals[2 * j + 1])
                    return carry3

                lax.fori_loop(0, _BCHUNK // _LANES, g_body, 0, unroll=2)

                _emit_tiles(buf, tile_row0, t * _BTILES, sems[par])
            return carry2

        lax.fori_loop(0, n_chunks // _NBUF, chunk_body, 0)
        for par in range(_NBUF):
            _drain_tiles(bufs[par], tile_row0, sems[par])
        return carry

    lax.fori_loop(u_start, u_end, unit_body, 0)


def kernel(inputs, table):
    b, h, w, ch = inputs.shape
    assert ch == _N_CHANNELS and table.shape == (_N_CHANNELS * _MAX_VALUE, _EMBED_DIM)
    n_units = h * w * ch
    idx_lin = jnp.transpose(inputs, (1, 2, 3, 0)).reshape(-1)
    table_flat = table.reshape(-1)
    n_btiles = b // 128
    n_tile_rows = n_units * _BANDS

    mesh = plsc.VectorSubcoreMesh(core_axis_name="c", subcore_axis_name="s")
    sc_call = pl.kernel(
        functools.partial(_sc_body, n_units, b),
        out_type=jax.ShapeDtypeStruct((n_tile_rows * n_btiles, 8, 128),
                                      jnp.float32),
        mesh=mesh,
        scratch_types=[
            pltpu.VMEM((table.size,), jnp.float32),
            pltpu.VMEM((b,), jnp.int32),
            pltpu.VMEM((_EMBED_DIM, _PITCH), jnp.float32),
            pltpu.VMEM((_EMBED_DIM, _PITCH), jnp.float32),
            pltpu.SemaphoreType.DMA,
            pltpu.SemaphoreType.DMA,
        ],
        compiler_params=pltpu.CompilerParams(
            use_tc_tiling_on_sc=False,
            needs_layout_passes=False,
        ),
    )
    out = sc_call(idx_lin, table_flat)
    t = out.reshape(h, w, ch * _BANDS, n_btiles, 8, 128)
    t = t.transpose(3, 5, 0, 1, 2, 4)
    return t.reshape(b, h, w, ch * _EMBED_DIM)
